# baseline (speedup 1.0000x reference)
import jax
import jax.numpy as jnp
from jax import lax
from jax.experimental import pallas as pl
from jax.experimental.pallas import tpu as pltpu
from jax.experimental.pallas import tpu_sc as plsc

NUM_VOCAB = 49408
NUM_EMBED = 768
NUM_TOKENS = 77
BATCH = 1024

_NW = 32
_ITEMS_PER_W = BATCH // _NW
_TPAD = 80
_T0 = (0, 16, 32, 48, 64)
_TN = (16, 16, 16, 16, 13)
_NBUF = len(_T0)
_LANES = 16
_CPL = NUM_EMBED // _LANES


def _add_pos(buf, pos_v, t0, nrows):
  def row_body(r, carry):
    for c in range(_CPL):
      sl = pl.ds(c * _LANES, _LANES)
      plsc.addupdate(buf.at[r, sl], pos_v[t0 + r, sl])
    return carry

  lax.fori_loop(0, nrows, row_body, 0, unroll=False)


def _embed_body(table_hbm, idx_hbm, pos_hbm, out_hbm,
                idx_v, pos_v, buf0, buf1, buf2, buf3, buf4,
                g0, g1, g2, g3, g4, s0, s1, s2, s3, s4):
  bufs = (buf0, buf1, buf2, buf3, buf4)
  gsems = (g0, g1, g2, g3, g4)
  ssems = (s0, s1, s2, s3, s4)

  wid = lax.axis_index("s") * 2 + lax.axis_index("c")
  item_base = wid * _ITEMS_PER_W

  pltpu.sync_copy(idx_hbm.at[pl.ds(item_base * _TPAD, _ITEMS_PER_W * _TPAD)],
                  idx_v)
  pltpu.sync_copy(pos_hbm, pos_v)

  def gather(i, j):
    off = pl.multiple_of(i * _TPAD + _T0[j], 8)
    return pltpu.async_copy(
        table_hbm.at[idx_v.at[pl.ds(off, _TN[j])]],
        bufs[j], gsems[j])

  def scatter(i, j):
    return pltpu.async_copy(
        bufs[j], out_hbm.at[item_base + i, pl.ds(_T0[j], _TN[j])], ssems[j])

  def drain_scatter(j):
    pltpu.make_async_copy(bufs[j], out_hbm.at[0, pl.ds(_T0[j], _TN[j])],
                          ssems[j]).wait()

  def item_body(i, carry):
    handles = []
    for j in range(_NBUF):
      @pl.when(i > 0)
      def _(j=j):
        drain_scatter(j)
      handles.append(gather(i, j))
    for j in range(_NBUF):
      handles[j].wait()
      _add_pos(bufs[j], pos_v, _T0[j], _TN[j])
      scatter(i, j)
    return carry

  lax.fori_loop(0, _ITEMS_PER_W, item_body, 0, unroll=False)

  for j in range(_NBUF):
    drain_scatter(j)


@jax.jit
def _embed(table, idx, pos):
  mesh = plsc.VectorSubcoreMesh(core_axis_name="c", subcore_axis_name="s",
                                num_cores=2, num_subcores=16)
  return pl.kernel(
      _embed_body,
      out_type=jax.ShapeDtypeStruct((BATCH, NUM_TOKENS, NUM_EMBED),
                                    jnp.float32),
      mesh=mesh,
      compiler_params=pltpu.CompilerParams(use_tc_tiling_on_sc=False),
      scratch_types=[
          pltpu.VMEM((_ITEMS_PER_W * _TPAD,), jnp.int32),
          pltpu.VMEM((NUM_TOKENS, NUM_EMBED), jnp.float32),
      ] + [pltpu.VMEM((n, NUM_EMBED), jnp.float32) for n in _TN]
        + [pltpu.SemaphoreType.DMA] * (2 * _NBUF),
  )(table, idx, pos)


def kernel(tokens, token_embedding, positional_embedding):
  idx = jnp.pad(tokens.astype(jnp.int32),
                ((0, 0), (0, _TPAD - NUM_TOKENS))).reshape(-1)
  out = _embed(token_embedding, idx, positional_embedding)
  patch = lax.slice(out, (0, 0, 0), (1, 1, 1))
  return lax.dynamic_update_slice(out, patch, (0, 0, 0))


# speedup vs baseline: 1.2105x; 1.2105x over previous
import jax
import jax.numpy as jnp
from jax import lax
from jax.experimental import pallas as pl
from jax.experimental.pallas import tpu as pltpu
from jax.experimental.pallas import tpu_sc as plsc

NUM_VOCAB = 49408
NUM_EMBED = 768
NUM_TOKENS = 77
BATCH = 1024

_NW = 32
_ITEMS_PER_W = BATCH // _NW
_TPAD = 80
_MAIN = 72
_TAIL = NUM_TOKENS - _MAIN
_LANES = 16
_CPL = NUM_EMBED // _LANES


def _embed_body(table_hbm, idx_hbm, pos_hbm, out_hbm,
                idx_v, pos_v, buf, tail_v, gsem, tsem, ssem):
  wid = lax.axis_index("s") * 2 + lax.axis_index("c")
  item_base = wid * _ITEMS_PER_W

  pltpu.sync_copy(pos_hbm, pos_v)

  def item_body(i, carry):
    pltpu.sync_copy(
        idx_hbm.at[pl.ds(pl.multiple_of((item_base + i) * _TPAD, 8), _TPAD)],
        idx_v)

    @pl.when(i > 0)
    def _():
      pltpu.make_async_copy(buf, out_hbm.at[0], ssem).wait()

    h_main = pltpu.async_copy(
        table_hbm.at[idx_v.at[pl.ds(0, _MAIN)]],
        buf.at[pl.ds(0, _MAIN)], gsem)
    h_tail = pltpu.async_copy(
        table_hbm.at[idx_v.at[pl.ds(_MAIN, 8)]],
        tail_v, tsem)

    h_tail.wait()

    def tail_body(r, carry2):
      for c in range(_CPL):
        sl = pl.ds(c * _LANES, _LANES)
        buf[_MAIN + r, sl] = tail_v[r, sl]
      return carry2

    lax.fori_loop(0, _TAIL, tail_body, 0, unroll=False)
    h_main.wait()

    def row_body(r, carry2):
      for c in range(_CPL):
        sl = pl.ds(c * _LANES, _LANES)
        plsc.addupdate(buf.at[r, sl], pos_v[r, sl])
      return carry2

    lax.fori_loop(0, NUM_TOKENS, row_body, 0, unroll=False)

    pltpu.async_copy(buf, out_hbm.at[item_base + i], ssem)
    return carry

  lax.fori_loop(0, _ITEMS_PER_W, item_body, 0, unroll=False)
  pltpu.make_async_copy(buf, out_hbm.at[0], ssem).wait()


@jax.jit
def _embed(table, idx, pos):
  mesh = plsc.VectorSubcoreMesh(core_axis_name="c", subcore_axis_name="s",
                                num_cores=2, num_subcores=16)
  return pl.kernel(
      _embed_body,
      out_type=jax.ShapeDtypeStruct((BATCH, NUM_TOKENS, NUM_EMBED),
                                    jnp.float32),
      mesh=mesh,
      scratch_types=[
          pltpu.VMEM((_TPAD,), jnp.int32),
          pltpu.VMEM((NUM_TOKENS, NUM_EMBED), jnp.float32),
          pltpu.VMEM((NUM_TOKENS, NUM_EMBED), jnp.float32),
          pltpu.VMEM((8, NUM_EMBED), jnp.float32),
          pltpu.SemaphoreType.DMA,
          pltpu.SemaphoreType.DMA,
          pltpu.SemaphoreType.DMA,
      ],
  )(table, idx, pos)


def kernel(tokens, token_embedding, positional_embedding):
  idx = jnp.pad(tokens.astype(jnp.int32),
                ((0, 0), (0, _TPAD - NUM_TOKENS))).reshape(-1)
  return _embed(token_embedding, idx, positional_embedding)
